# baseline (speedup 1.0000x reference)
import dataclasses
import functools

import jax
import jax.numpy as jnp
from jax import lax
from jax.experimental import pallas as pl
from jax.experimental.pallas import tpu as pltpu
from jax.experimental.pallas import tpu_sc as plsc

_SMOOTHING = 0.1


def _sc_gather_sums(xt, t):
    n = t.shape[0]
    n_cols = xt.shape[1]
    mesh = plsc.VectorSubcoreMesh(core_axis_name="c", subcore_axis_name="s")
    n_sub = 16
    per = n // (2 * n_sub)
    lanes = 16

    cp = pltpu.CompilerParams()
    if "needs_layout_passes" in pltpu.CompilerParams.__dataclass_fields__:
        cp = dataclasses.replace(cp, needs_layout_passes=False)

    @functools.partial(
        pl.kernel,
        out_type=jax.ShapeDtypeStruct((2 * n_sub, lanes), jnp.float32),
        mesh=mesh,
        compiler_params=cp,
        scratch_types=[
            pltpu.VMEM((per,), jnp.int32),
            pltpu.VMEM((per, n_cols), jnp.float32),
            pltpu.VMEM((lanes,), jnp.float32),
        ],
    )
    def gather_kernel(xt_hbm, t_hbm, o_hbm, t_v, rows_v, acc_v):
        cid = lax.axis_index("c")
        sid = lax.axis_index("s")
        wid = cid * n_sub + sid
        base = wid * per
        pltpu.sync_copy(t_hbm.at[pl.ds(base, per)], t_v)
        pltpu.sync_copy(xt_hbm.at[t_v], rows_v)
        iota = lax.iota(jnp.int32, lanes)
        acc = jnp.zeros((lanes,), jnp.float32)
        for g in range(per // lanes):
            j_idx = iota + g * lanes
            col_idx = j_idx + base
            acc = acc + plsc.load_gather(rows_v, [j_idx, col_idx])
        acc_v[...] = jnp.full((lanes,), jnp.sum(acc), jnp.float32)
        pltpu.sync_copy(acc_v, o_hbm.at[wid])

    return gather_kernel(xt, t)


def _tc_sum(xt):
    n_cols, n_rows = xt.shape
    br = 800
    steps = n_cols // br
    nb = 10

    def body(x_hbm, o_ref, bufs, sems):
        def mk(i, b):
            return pltpu.make_async_copy(
                x_hbm.at[pl.ds(i * br, br), :], bufs.at[b], sems.at[b]
            )

        for b in range(nb):
            mk(b, b).start()
        s = jnp.float32(0.0)
        for i in range(steps):
            b = i % nb
            mk(i, b).wait()
            s = s + jnp.sum(bufs[b])
            if i + nb < steps:
                mk(i + nb, b).start()
        o_ref[0] = s

    return pl.pallas_call(
        body,
        in_specs=[pl.BlockSpec(memory_space=pl.ANY)],
        out_specs=pl.BlockSpec(memory_space=pltpu.SMEM),
        out_shape=jax.ShapeDtypeStruct((1,), jnp.float32),
        scratch_shapes=[
            pltpu.VMEM((nb, br, n_rows), jnp.float32),
            pltpu.SemaphoreType.DMA((nb,)),
        ],
        compiler_params=pltpu.CompilerParams(vmem_limit_bytes=50 * 1024 * 1024),
    )(xt)


def kernel(input, target):
    n_rows, n_cols = input.shape
    t32 = target.astype(jnp.int32)
    xt = input.T
    tsums = _tc_sum(xt)
    lsums = _sc_gather_sums(xt, t32)
    l_sum = jnp.sum(lsums[:, 0])
    total = tsums[0]
    return (_SMOOTHING - 1.0) * l_sum - _SMOOTHING * total / (n_rows * n_cols)


# speedup vs baseline: 1.0121x; 1.0121x over previous
import dataclasses
import functools

import jax
import jax.numpy as jnp
from jax import lax
from jax.experimental import pallas as pl
from jax.experimental.pallas import tpu as pltpu
from jax.experimental.pallas import tpu_sc as plsc

_SMOOTHING = 0.1


def _sc_gather_sums(xt, t):
    n = t.shape[0]
    n_cols = xt.shape[1]
    mesh = plsc.VectorSubcoreMesh(core_axis_name="c", subcore_axis_name="s")
    n_sub = 16
    per = n // (2 * n_sub)
    lanes = 16

    cp = pltpu.CompilerParams()
    if "needs_layout_passes" in pltpu.CompilerParams.__dataclass_fields__:
        cp = dataclasses.replace(cp, needs_layout_passes=False)

    @functools.partial(
        pl.kernel,
        out_type=jax.ShapeDtypeStruct((2 * n_sub, lanes), jnp.float32),
        mesh=mesh,
        compiler_params=cp,
        scratch_types=[
            pltpu.VMEM((per,), jnp.int32),
            pltpu.VMEM((per, n_cols), jnp.float32),
            pltpu.VMEM((lanes,), jnp.float32),
        ],
    )
    def gather_kernel(xt_hbm, t_hbm, o_hbm, t_v, rows_v, acc_v):
        cid = lax.axis_index("c")
        sid = lax.axis_index("s")
        wid = cid * n_sub + sid
        base = wid * per
        pltpu.sync_copy(t_hbm.at[pl.ds(base, per)], t_v)
        pltpu.sync_copy(xt_hbm.at[t_v], rows_v)
        iota = lax.iota(jnp.int32, lanes)
        acc = jnp.zeros((lanes,), jnp.float32)
        for g in range(per // lanes):
            j_idx = iota + g * lanes
            col_idx = j_idx + base
            acc = acc + plsc.load_gather(rows_v, [j_idx, col_idx])
        acc_v[...] = jnp.full((lanes,), jnp.sum(acc), jnp.float32)
        pltpu.sync_copy(acc_v, o_hbm.at[wid])

    return gather_kernel(xt, t)


def _tc_sum(xt):
    n_cols, n_rows = xt.shape
    br = 1000
    steps = n_cols // br
    nb = 8

    def body(x_hbm, o_ref, bufs, sems):
        def mk(i, b):
            return pltpu.make_async_copy(
                x_hbm.at[pl.ds(i * br, br), :], bufs.at[b], sems.at[b]
            )

        for b in range(nb):
            mk(b, b).start()
        s = jnp.float32(0.0)
        for i in range(steps):
            b = i % nb
            mk(i, b).wait()
            s = s + jnp.sum(bufs[b])
            if i + nb < steps:
                mk(i + nb, b).start()
        o_ref[0] = s

    return pl.pallas_call(
        body,
        in_specs=[pl.BlockSpec(memory_space=pl.ANY)],
        out_specs=pl.BlockSpec(memory_space=pltpu.SMEM),
        out_shape=jax.ShapeDtypeStruct((1,), jnp.float32),
        scratch_shapes=[
            pltpu.VMEM((nb, br, n_rows), jnp.float32),
            pltpu.SemaphoreType.DMA((nb,)),
        ],
        compiler_params=pltpu.CompilerParams(vmem_limit_bytes=50 * 1024 * 1024),
    )(xt)


def kernel(input, target):
    n_rows, n_cols = input.shape
    t32 = target.astype(jnp.int32)
    xt = input.T
    tsums = _tc_sum(xt)
    lsums = _sc_gather_sums(xt, t32)
    l_sum = jnp.sum(lsums[:, 0])
    total = tsums[0]
    return (_SMOOTHING - 1.0) * l_sum - _SMOOTHING * total / (n_rows * n_cols)
